# baseline (speedup 1.0000x reference)
import functools

import jax
import jax.numpy as jnp
from jax import lax
from jax.experimental import pallas as pl
from jax.experimental.pallas import tpu as pltpu
from jax.experimental.pallas import tpu_sc as plsc

_NUM_EMBED = 1024
_DIM = 64
_COMMIT = 0.25
_INTERPRET = False


def _dist_body(z_ref, w_ref, idx_ref, loss_ref, wpad_ref, *, scale, tile,
               cblk):
    z = z_ref[...].reshape(tile, _DIM)
    w = w_ref[...]
    zsq = jnp.sum(z * z, axis=1, keepdims=True)
    wsq = jnp.sum(w * w, axis=1)
    z2 = z + z
    fbig = float(_NUM_EMBED)
    dmins, idxs = [], []
    for cb in range(_NUM_EMBED // cblk):
        wb = w[cb * cblk:(cb + 1) * cblk, :]
        mm2 = jax.lax.dot_general(z2, wb, (((1,), (1,)), ((), ())),
                                  preferred_element_type=jnp.float32)
        db = zsq + wsq[None, cb * cblk:(cb + 1) * cblk] - mm2
        dmin_b = jnp.min(db, axis=1, keepdims=True)
        col = (jax.lax.broadcasted_iota(jnp.int32, db.shape, 1)
               .astype(jnp.float32) + float(cb * cblk))
        idx_b = jnp.min(jnp.where(db == dmin_b, col, fbig), axis=1,
                        keepdims=True)
        dmins.append(dmin_b)
        idxs.append(idx_b)
    dmin = functools.reduce(jnp.minimum, dmins)
    idx_f = functools.reduce(
        jnp.minimum,
        [jnp.where(db == dmin, ib, fbig) for db, ib in zip(dmins, idxs)])
    idx = idx_f[:, 0].astype(jnp.int32)
    idx_ref[pl.ds(pl.program_id(0) * tile, tile)] = idx
    part = scale * jnp.sum(dmin)

    @pl.when(pl.program_id(0) == 0)
    def _init():
        loss_ref[0, 0] = 0.0
        wpad_ref[:, :_DIM] = w
        wpad_ref[:, _DIM:] = jnp.zeros((_NUM_EMBED, _DIM), jnp.float32)

    loss_ref[0, 0] += part


def _dist_argmin(inputs, weight, b_blk, total_n=None):
    b, s, _ = inputs.shape
    n = b * s
    tile = b_blk * s
    grid = b // b_blk
    scale = (1.0 + _COMMIT) / ((total_n or n) * _DIM)
    idx1, loss, wpad = pl.pallas_call(
        functools.partial(_dist_body, scale=scale, tile=tile, cblk=512),
        grid=(grid,),
        in_specs=[
            pl.BlockSpec((b_blk, s, _DIM), lambda i: (i, 0, 0)),
            pl.BlockSpec((_NUM_EMBED, _DIM), lambda i: (0, 0)),
        ],
        out_specs=[
            pl.BlockSpec((n,), lambda i: (0,)),
            pl.BlockSpec(memory_space=pltpu.SMEM, block_shape=(1, 1),
                         index_map=lambda i: (0, 0)),
            pl.BlockSpec((_NUM_EMBED, 2 * _DIM), lambda i: (0, 0)),
        ],
        out_shape=[
            jax.ShapeDtypeStruct((n,), jnp.int32),
            jax.ShapeDtypeStruct((1, 1), jnp.float32),
            jax.ShapeDtypeStruct((_NUM_EMBED, 2 * _DIM), jnp.float32),
        ],
        interpret=_INTERPRET,
    )(inputs, weight)
    return idx1, loss[0, 0], wpad


_NC = 2
_NS = 16
_NW = _NC * _NS


def _make_sc_gather(n, chunk):
    rows_per_w = n // _NW
    nchunk = rows_per_w // chunk
    mesh = plsc.VectorSubcoreMesh(core_axis_name="c", subcore_axis_name="s",
                                  num_cores=_NC, num_subcores=_NS)

    @functools.partial(
        pl.kernel, mesh=mesh,
        out_type=jax.ShapeDtypeStruct((n, 2 * _DIM), jnp.float32),
        scratch_types=[
            pltpu.VMEM((rows_per_w,), jnp.int32),
            pltpu.VMEM((rows_per_w, 2 * _DIM), jnp.float32),
            pltpu.SemaphoreType.DMA,
        ],
    )
    def sc_gather(table_hbm, idx_hbm, out_hbm, idx_v, rows_v, sem):
        wid = lax.axis_index("s") * _NC + lax.axis_index("c")
        base = wid * rows_per_w
        pltpu.sync_copy(idx_hbm.at[pl.ds(base, rows_per_w)], idx_v)
        copies = []
        for j in range(nchunk):
            copies.append(pltpu.async_copy(
                table_hbm.at[idx_v.at[pl.ds(j * chunk, chunk)]],
                rows_v.at[pl.ds(j * chunk, chunk)], sem))
        for c in copies:
            c.wait()
        pltpu.sync_copy(rows_v, out_hbm.at[pl.ds(base, rows_per_w)])

    return sc_gather


@jax.jit
def _vq_forward(inputs, weight):
    b, s, dim = inputs.shape
    n = b * s
    half = b // 2
    nh = half * s
    idx_a, loss_a, wpad = _dist_argmin(inputs[:half], weight, b_blk=4,
                                       total_n=n)
    idx_b, loss_b, _ = _dist_argmin(inputs[half:], weight, b_blk=4,
                                    total_n=n)
    gather = _make_sc_gather(nh, chunk=96)
    qst_a = gather(wpad, idx_a)
    qst_b = gather(wpad, idx_b)
    qst = jnp.concatenate([qst_a[:, :_DIM], qst_b[:, :_DIM]], axis=0)
    idx = jnp.concatenate([idx_a, idx_b])
    return qst, idx, loss_a + loss_b


def kernel(inputs, embedding_weight):
    b, s, dim = inputs.shape
    qst, idx, vq_loss = _vq_forward(inputs, embedding_weight)
    return qst.reshape(inputs.shape), vq_loss, idx.reshape(b, s)


# speedup vs baseline: 1.1165x; 1.1165x over previous
import functools

import jax
import jax.numpy as jnp
from jax import lax
from jax.experimental import pallas as pl
from jax.experimental.pallas import tpu as pltpu
from jax.experimental.pallas import tpu_sc as plsc

_NUM_EMBED = 1024
_DIM = 64
_COMMIT = 0.25
_INTERPRET = False


def _dist_body(z_ref, w_ref, idx_ref, loss_ref, wpad_ref, *, scale, tile,
               cblk):
    z = z_ref[...].reshape(tile, _DIM)
    w = w_ref[...]
    zsq = jnp.sum(z * z, axis=1, keepdims=True)
    wsq = jnp.sum(w * w, axis=1)
    z2 = z + z
    fbig = float(_NUM_EMBED)
    dmins, idxs = [], []
    for cb in range(_NUM_EMBED // cblk):
        wb = w[cb * cblk:(cb + 1) * cblk, :]
        mm2 = jax.lax.dot_general(z2, wb, (((1,), (1,)), ((), ())),
                                  preferred_element_type=jnp.float32)
        db = zsq + wsq[None, cb * cblk:(cb + 1) * cblk] - mm2
        dmin_b = jnp.min(db, axis=1, keepdims=True)
        col = (jax.lax.broadcasted_iota(jnp.int32, db.shape, 1)
               .astype(jnp.float32) + float(cb * cblk))
        idx_b = jnp.min(jnp.where(db == dmin_b, col, fbig), axis=1,
                        keepdims=True)
        dmins.append(dmin_b)
        idxs.append(idx_b)
    dmin = functools.reduce(jnp.minimum, dmins)
    idx_f = functools.reduce(
        jnp.minimum,
        [jnp.where(db == dmin, ib, fbig) for db, ib in zip(dmins, idxs)])
    idx = idx_f[:, 0].astype(jnp.int32)
    idx_ref[pl.ds(pl.program_id(0) * tile, tile)] = idx
    part = scale * jnp.sum(dmin)

    @pl.when(pl.program_id(0) == 0)
    def _init():
        loss_ref[0, 0] = 0.0
        wpad_ref[:, :_DIM] = w
        wpad_ref[:, _DIM:] = jnp.zeros((_NUM_EMBED, _DIM), jnp.float32)

    loss_ref[0, 0] += part


def _dist_argmin(inputs, weight, b_blk):
    b, s, _ = inputs.shape
    n = b * s
    tile = b_blk * s
    grid = b // b_blk
    scale = (1.0 + _COMMIT) / (n * _DIM)
    idx1, loss, wpad = pl.pallas_call(
        functools.partial(_dist_body, scale=scale, tile=tile, cblk=512),
        grid=(grid,),
        in_specs=[
            pl.BlockSpec((b_blk, s, _DIM), lambda i: (i, 0, 0)),
            pl.BlockSpec((_NUM_EMBED, _DIM), lambda i: (0, 0)),
        ],
        out_specs=[
            pl.BlockSpec((n,), lambda i: (0,)),
            pl.BlockSpec(memory_space=pltpu.SMEM, block_shape=(1, 1),
                         index_map=lambda i: (0, 0)),
            pl.BlockSpec((_NUM_EMBED, 2 * _DIM), lambda i: (0, 0)),
        ],
        out_shape=[
            jax.ShapeDtypeStruct((n,), jnp.int32),
            jax.ShapeDtypeStruct((1, 1), jnp.float32),
            jax.ShapeDtypeStruct((_NUM_EMBED, 2 * _DIM), jnp.float32),
        ],
        interpret=_INTERPRET,
    )(inputs, weight)
    return idx1, loss[0, 0], wpad


_NC = 2
_NS = 16
_NW = _NC * _NS


def _make_sc_gather(n, chunk):
    rows_per_w = n // _NW
    nchunk = rows_per_w // chunk
    mesh = plsc.VectorSubcoreMesh(core_axis_name="c", subcore_axis_name="s",
                                  num_cores=_NC, num_subcores=_NS)

    @functools.partial(
        pl.kernel, mesh=mesh,
        out_type=jax.ShapeDtypeStruct((n, 2 * _DIM), jnp.float32),
        scratch_types=[
            pltpu.VMEM((rows_per_w,), jnp.int32),
            pltpu.VMEM((rows_per_w, 2 * _DIM), jnp.float32),
            pltpu.SemaphoreType.DMA,
        ],
    )
    def sc_gather(table_hbm, idx_hbm, out_hbm, idx_v, rows_v, sem):
        wid = lax.axis_index("s") * _NC + lax.axis_index("c")
        base = wid * rows_per_w
        pltpu.sync_copy(idx_hbm.at[pl.ds(base, rows_per_w)], idx_v)
        copies = []
        for j in range(nchunk):
            copies.append(pltpu.async_copy(
                table_hbm.at[idx_v.at[pl.ds(j * chunk, chunk)]],
                rows_v.at[pl.ds(j * chunk, chunk)], sem))
        for c in copies:
            c.wait()
        pltpu.sync_copy(rows_v, out_hbm.at[pl.ds(base, rows_per_w)])

    return sc_gather


@jax.jit
def _vq_forward(inputs, weight):
    b, s, dim = inputs.shape
    n = b * s
    idx, vq_loss, wpad = _dist_argmin(inputs, weight, b_blk=4)
    qst = _make_sc_gather(n, chunk=96)(wpad, idx)[:, :_DIM]
    return qst, idx, vq_loss


def kernel(inputs, embedding_weight):
    b, s, dim = inputs.shape
    qst, idx, vq_loss = _vq_forward(inputs, embedding_weight)
    return qst.reshape(inputs.shape), vq_loss, idx.reshape(b, s)
